# Initial kernel scaffold; baseline (speedup 1.0000x reference)
import jax
import jax.numpy as jnp
from jax.experimental import pallas as pl


def kernel(input_matrix, adjacency_coo_matrix, weights_matrix):
    raise NotImplementedError("write your pallas kernel here")


# speedup vs baseline: 10.0460x; 10.0460x over previous
import functools

import jax
import jax.numpy as jnp
from jax import lax
from jax.experimental import pallas as pl
from jax.experimental.pallas import tpu as pltpu
from jax.experimental.pallas import tpu_sc as plsc

N = 10000
D = 128
NROWS = 10240
ROWS_PER_TILE = NROWS // 16
CHUNK = 128
NCHUNK = 79
PER_TILE = NCHUNK * CHUNK
NTILES = 32


def _sc_aggregate(x, src_p, dst_p):
  mesh = plsc.VectorSubcoreMesh(core_axis_name="c", subcore_axis_name="s")

  @functools.partial(
      pl.kernel,
      out_type=(
          jax.ShapeDtypeStruct((2, NROWS, D), jnp.float32),
          jax.ShapeDtypeStruct((2, NROWS), jnp.float32),
      ),
      mesh=mesh,
      scratch_types=[
          pltpu.VMEM((CHUNK,), jnp.int32),
          pltpu.VMEM((CHUNK,), jnp.int32),
          pltpu.VMEM((CHUNK, D), jnp.float32),
          pltpu.VMEM((CHUNK,), jnp.float32),
          pltpu.VMEM((ROWS_PER_TILE,), jnp.float32),
          pltpu.VMEM_SHARED((NROWS, D), jnp.float32),
          pltpu.VMEM_SHARED((NROWS,), jnp.float32),
          pltpu.SemaphoreType.DMA,
      ],
  )
  def agg(x_hbm, src_hbm, dst_hbm, sums_hbm, counts_hbm,
          src_idx, dst_idx, rows, ones_v, cbuf, acc, cacc, sem):
    c = lax.axis_index("c")
    s = lax.axis_index("s")
    wid = s * 2 + c
    zbase = s * ROWS_PER_TILE
    z16 = jnp.zeros((16,), jnp.float32)
    one16 = jnp.ones((16,), jnp.float32)

    def zrow(r, _):
      def zcol(j, _):
        rows[r, pl.ds(j * 16, 16)] = z16
        return 0
      return lax.fori_loop(0, D // 16, zcol, 0)
    lax.fori_loop(0, CHUNK, zrow, 0)
    def zone(j, _):
      ones_v[pl.ds(j * 16, 16)] = one16
      return 0
    lax.fori_loop(0, CHUNK // 16, zone, 0)
    def zcbuf(j, _):
      cbuf[pl.ds(j * 16, 16)] = z16
      return 0
    lax.fori_loop(0, ROWS_PER_TILE // 16, zcbuf, 0)
    def zcopy(k, _):
      pltpu.sync_copy(rows, acc.at[pl.ds(zbase + k * CHUNK, CHUNK)])
      return 0
    lax.fori_loop(0, ROWS_PER_TILE // CHUNK, zcopy, 0)
    pltpu.sync_copy(cbuf, cacc.at[pl.ds(zbase, ROWS_PER_TILE)])
    plsc.subcore_barrier()

    def chunk_body(j, _):
      pltpu.sync_copy(src_hbm.at[wid, j], src_idx)
      pltpu.sync_copy(dst_hbm.at[wid, j], dst_idx)
      pltpu.async_copy(x_hbm.at[dst_idx], rows, sem).wait()
      pltpu.sync_copy(rows, acc.at[src_idx], add=True)
      pltpu.sync_copy(ones_v, cacc.at[src_idx], add=True)
      return 0
    lax.fori_loop(0, NCHUNK, chunk_body, 0)
    plsc.subcore_barrier()

    def ecopy(k, _):
      r = zbase + k * CHUNK
      pltpu.sync_copy(acc.at[pl.ds(r, CHUNK)], rows)
      pltpu.sync_copy(rows, sums_hbm.at[c, pl.ds(r, CHUNK)])
      return 0
    lax.fori_loop(0, ROWS_PER_TILE // CHUNK, ecopy, 0)
    pltpu.sync_copy(cacc.at[pl.ds(zbase, ROWS_PER_TILE)], cbuf)
    pltpu.sync_copy(cbuf, counts_hbm.at[c, pl.ds(zbase, ROWS_PER_TILE)])

  return agg(x, src_p, dst_p)


def _tc_finish_body(p0_ref, p1_ref, c0_ref, c1_ref, x_ref, w_ref, o_ref):
  cnt = c0_ref[...] + c1_ref[...] + 1.0
  mean = (p0_ref[...] + p1_ref[...] + x_ref[...]) / cnt
  o_ref[...] = jnp.dot(mean, w_ref[...], preferred_element_type=jnp.float32)


def _tc_finish(p0, p1, c0, c1, x, w):
  blk = 1000
  grid = N // blk
  return pl.pallas_call(
      _tc_finish_body,
      grid=(grid,),
      in_specs=[
          pl.BlockSpec((blk, D), lambda i: (i, 0)),
          pl.BlockSpec((blk, D), lambda i: (i, 0)),
          pl.BlockSpec((blk, 1), lambda i: (i, 0)),
          pl.BlockSpec((blk, 1), lambda i: (i, 0)),
          pl.BlockSpec((blk, D), lambda i: (i, 0)),
          pl.BlockSpec((D, D), lambda i: (0, 0)),
      ],
      out_specs=pl.BlockSpec((blk, D), lambda i: (i, 0)),
      out_shape=jax.ShapeDtypeStruct((N, D), jnp.float32),
  )(p0, p1, c0, c1, x, w)


def kernel(input_matrix, adjacency_coo_matrix, weights_matrix):
  x = input_matrix
  src = adjacency_coo_matrix[0].astype(jnp.int32)
  dst = adjacency_coo_matrix[1].astype(jnp.int32)
  e = src.shape[0]
  total = NTILES * PER_TILE
  npad = total - e
  pad_ar = jnp.arange(npad, dtype=jnp.int32)
  src_p = jnp.concatenate([src, N + (pad_ar % 16)]).reshape(NTILES, NCHUNK, CHUNK)
  dst_p = jnp.concatenate([dst, (pad_ar * 97) % N]).reshape(NTILES, NCHUNK, CHUNK)

  sums, counts = _sc_aggregate(x, src_p, dst_p)
  p0 = sums[0, :N]
  p1 = sums[1, :N]
  c0 = counts[0, :N].reshape(N, 1)
  c1 = counts[1, :N].reshape(N, 1)
  return _tc_finish(p0, p1, c0, c1, x, weights_matrix)


# speedup vs baseline: 16.8371x; 1.6760x over previous
import functools

import jax
import jax.numpy as jnp
from jax import lax
from jax.experimental import pallas as pl
from jax.experimental.pallas import tpu as pltpu
from jax.experimental.pallas import tpu_sc as plsc

N = 10000
D = 128
NROWS = 10240
ROWS_PER_TILE = NROWS // 16
CHUNK = 128
NCHUNK = 79
PER_TILE = NCHUNK * CHUNK
NTILES = 32
NBUF = 4


def _sc_aggregate(x, idx_p):
  mesh = plsc.VectorSubcoreMesh(core_axis_name="c", subcore_axis_name="s")

  @functools.partial(
      pl.kernel,
      out_type=(
          jax.ShapeDtypeStruct((2, NROWS, D), jnp.float32),
          jax.ShapeDtypeStruct((2, NROWS), jnp.float32),
      ),
      mesh=mesh,
      scratch_types=[
          pltpu.VMEM((2, 2, CHUNK), jnp.int32),
          pltpu.VMEM((2, CHUNK, D), jnp.float32),
          pltpu.VMEM((CHUNK,), jnp.float32),
          pltpu.VMEM((ROWS_PER_TILE,), jnp.float32),
          pltpu.VMEM_SHARED((NROWS, D), jnp.float32),
          pltpu.VMEM_SHARED((NROWS,), jnp.float32),
          pltpu.SemaphoreType.DMA((2,)),
          pltpu.SemaphoreType.DMA((2,)),
      ],
  )
  def agg(x_hbm, idx_hbm, sums_hbm, counts_hbm,
          idx_v, rows, ones_v, cbuf, acc, cacc, isem, gsem):
    c = lax.axis_index("c")
    s = lax.axis_index("s")
    wid = s * 2 + c
    zbase = s * ROWS_PER_TILE
    z16 = jnp.zeros((16,), jnp.float32)
    one16 = jnp.ones((16,), jnp.float32)

    def zrow(r, _):
      def zcol(j, _):
        rows[0, r, pl.ds(j * 16, 16)] = z16
        return 0
      return lax.fori_loop(0, D // 16, zcol, 0)
    lax.fori_loop(0, CHUNK, zrow, 0)
    def zone(j, _):
      ones_v[pl.ds(j * 16, 16)] = one16
      return 0
    lax.fori_loop(0, CHUNK // 16, zone, 0)
    def zcbuf(j, _):
      cbuf[pl.ds(j * 16, 16)] = z16
      return 0
    lax.fori_loop(0, ROWS_PER_TILE // 16, zcbuf, 0)
    def zcopy(k, _):
      pltpu.sync_copy(rows.at[0], acc.at[pl.ds(zbase + k * CHUNK, CHUNK)])
      return 0
    lax.fori_loop(0, ROWS_PER_TILE // CHUNK, zcopy, 0)
    pltpu.sync_copy(cbuf, cacc.at[pl.ds(zbase, ROWS_PER_TILE)])
    plsc.subcore_barrier()

    pltpu.async_copy(idx_hbm.at[wid, 0], idx_v.at[0], isem.at[0])
    pltpu.async_copy(idx_hbm.at[wid, 1], idx_v.at[1], isem.at[1])
    pltpu.make_async_copy(idx_hbm.at[wid, 0], idx_v.at[0], isem.at[0]).wait()
    pltpu.async_copy(x_hbm.at[idx_v.at[0, 1]], rows.at[0], gsem.at[0])
    def chunk_body(j, _):
      b = lax.rem(j, 2)
      b1 = lax.rem(j + 1, 2)
      @pl.when(j + 1 < NCHUNK)
      def _():
        pltpu.make_async_copy(idx_hbm.at[wid, j + 1], idx_v.at[b1],
                              isem.at[b1]).wait()
        pltpu.async_copy(x_hbm.at[idx_v.at[b1, 1]], rows.at[b1], gsem.at[b1])
      pltpu.make_async_copy(x_hbm.at[idx_v.at[b, 1]], rows.at[b],
                            gsem.at[b]).wait()
      pltpu.sync_copy(rows.at[b], acc.at[idx_v.at[b, 0]], add=True)
      pltpu.sync_copy(ones_v, cacc.at[idx_v.at[b, 0]], add=True)
      @pl.when(j + 2 < NCHUNK)
      def _():
        pltpu.async_copy(idx_hbm.at[wid, j + 2], idx_v.at[b], isem.at[b])
      return 0
    lax.fori_loop(0, NCHUNK, chunk_body, 0)
    plsc.subcore_barrier()

    def ecopy(k, _):
      r = zbase + k * CHUNK
      pltpu.sync_copy(acc.at[pl.ds(r, CHUNK)], rows.at[0])
      pltpu.sync_copy(rows.at[0], sums_hbm.at[c, pl.ds(r, CHUNK)])
      return 0
    lax.fori_loop(0, ROWS_PER_TILE // CHUNK, ecopy, 0)
    pltpu.sync_copy(cacc.at[pl.ds(zbase, ROWS_PER_TILE)], cbuf)
    pltpu.sync_copy(cbuf, counts_hbm.at[c, pl.ds(zbase, ROWS_PER_TILE)])

  return agg(x, idx_p)


def _tc_finish_body(p0_ref, p1_ref, c0_ref, c1_ref, x_ref, w_ref, o_ref):
  cnt = c0_ref[...] + c1_ref[...] + 1.0
  mean = (p0_ref[...] + p1_ref[...] + x_ref[...]) / cnt
  o_ref[...] = jnp.dot(mean, w_ref[...], preferred_element_type=jnp.float32)


def _tc_finish(p0, p1, c0, c1, x, w):
  blk = 1000
  grid = N // blk
  return pl.pallas_call(
      _tc_finish_body,
      grid=(grid,),
      in_specs=[
          pl.BlockSpec((blk, D), lambda i: (i, 0)),
          pl.BlockSpec((blk, D), lambda i: (i, 0)),
          pl.BlockSpec((blk, 1), lambda i: (i, 0)),
          pl.BlockSpec((blk, 1), lambda i: (i, 0)),
          pl.BlockSpec((blk, D), lambda i: (i, 0)),
          pl.BlockSpec((D, D), lambda i: (0, 0)),
      ],
      out_specs=pl.BlockSpec((blk, D), lambda i: (i, 0)),
      out_shape=jax.ShapeDtypeStruct((N, D), jnp.float32),
  )(p0, p1, c0, c1, x, w)


def kernel(input_matrix, adjacency_coo_matrix, weights_matrix):
  x = input_matrix
  src = adjacency_coo_matrix[0].astype(jnp.int32)
  dst = adjacency_coo_matrix[1].astype(jnp.int32)
  e = src.shape[0]
  total = NTILES * PER_TILE
  npad = total - e
  pad_ar = jnp.arange(npad, dtype=jnp.int32)
  src_p = jnp.concatenate([src, N + (pad_ar % 16)]).reshape(NTILES, NCHUNK, CHUNK)
  dst_p = jnp.concatenate([dst, (pad_ar * 97) % N]).reshape(NTILES, NCHUNK, CHUNK)
  idx_p = jnp.stack([src_p, dst_p], axis=2)

  sums, counts = _sc_aggregate(x, idx_p)
  p0 = sums[0, :N]
  p1 = sums[1, :N]
  c0 = counts[0, :N].reshape(N, 1)
  c1 = counts[1, :N].reshape(N, 1)
  return _tc_finish(p0, p1, c0, c1, x, weights_matrix)


# speedup vs baseline: 19.4276x; 1.1539x over previous
import functools

import jax
import jax.numpy as jnp
from jax import lax
from jax.experimental import pallas as pl
from jax.experimental.pallas import tpu as pltpu
from jax.experimental.pallas import tpu_sc as plsc

N = 10000
D = 128
NROWS = 10240
ROWS_PER_TILE = NROWS // 16
CHUNK = 128
NCHUNK = 79
PER_TILE = NCHUNK * CHUNK
NTILES = 32
NBUF = 4


def _sc_aggregate(x, idx_p):
  mesh = plsc.VectorSubcoreMesh(core_axis_name="c", subcore_axis_name="s")

  @functools.partial(
      pl.kernel,
      out_type=(
          jax.ShapeDtypeStruct((2, NROWS, D), jnp.float32),
          jax.ShapeDtypeStruct((2, NROWS), jnp.float32),
      ),
      mesh=mesh,
      scratch_types=[
          pltpu.VMEM((4, 2, CHUNK), jnp.int32),
          pltpu.VMEM((2, CHUNK, D), jnp.float32),
          pltpu.VMEM((CHUNK,), jnp.float32),
          pltpu.VMEM((ROWS_PER_TILE,), jnp.float32),
          pltpu.VMEM_SHARED((NROWS, D), jnp.float32),
          pltpu.VMEM_SHARED((NROWS,), jnp.float32),
          pltpu.SemaphoreType.DMA((4,)),
          pltpu.SemaphoreType.DMA((2,)),
          pltpu.SemaphoreType.DMA((2,)),
          pltpu.SemaphoreType.DMA((2,)),
      ],
  )
  def agg(x_hbm, idx_hbm, sums_hbm, counts_hbm,
          idx_v, rows, ones_v, cbuf, acc, cacc, isem, gsem, ssem, osem):
    c = lax.axis_index("c")
    s = lax.axis_index("s")
    wid = s * 2 + c
    zbase = s * ROWS_PER_TILE
    z16 = jnp.zeros((16,), jnp.float32)
    one16 = jnp.ones((16,), jnp.float32)

    def zrow(r, _):
      def zcol(j, _):
        rows[0, r, pl.ds(j * 16, 16)] = z16
        return 0
      return lax.fori_loop(0, D // 16, zcol, 0)
    lax.fori_loop(0, CHUNK, zrow, 0)
    def zone(j, _):
      ones_v[pl.ds(j * 16, 16)] = one16
      return 0
    lax.fori_loop(0, CHUNK // 16, zone, 0)
    def zcbuf(j, _):
      cbuf[pl.ds(j * 16, 16)] = z16
      return 0
    lax.fori_loop(0, ROWS_PER_TILE // 16, zcbuf, 0)
    def zcopy(k, _):
      pltpu.sync_copy(rows.at[0], acc.at[pl.ds(zbase + k * CHUNK, CHUNK)])
      return 0
    lax.fori_loop(0, ROWS_PER_TILE // CHUNK, zcopy, 0)
    pltpu.sync_copy(cbuf, cacc.at[pl.ds(zbase, ROWS_PER_TILE)])
    plsc.subcore_barrier()

    def idx_start(j):
      pltpu.async_copy(idx_hbm.at[wid, j], idx_v.at[lax.rem(j, 4)],
                       isem.at[lax.rem(j, 4)])
    def idx_wait(j):
      jb = lax.rem(j, 4)
      pltpu.make_async_copy(idx_hbm.at[wid, j], idx_v.at[jb],
                            isem.at[jb]).wait()
    def gather_start(j, b):
      pltpu.async_copy(x_hbm.at[idx_v.at[lax.rem(j, 4), 1]], rows.at[b],
                       gsem.at[b])
    def gather_wait(j, b):
      pltpu.make_async_copy(x_hbm.at[idx_v.at[lax.rem(j, 4), 1]], rows.at[b],
                            gsem.at[b]).wait()
    def scatter_start(j, b):
      jb = lax.rem(j, 4)
      pltpu.async_copy(rows.at[b], acc.at[idx_v.at[jb, 0]], ssem.at[b],
                       add=True)
      pltpu.async_copy(ones_v, cacc.at[idx_v.at[jb, 0]], osem.at[b],
                       add=True)
    def scatter_wait(j, b):
      jb = lax.rem(j, 4)
      pltpu.make_async_copy(rows.at[b], acc.at[idx_v.at[jb, 0]],
                            ssem.at[b]).wait()
      pltpu.make_async_copy(ones_v, cacc.at[idx_v.at[jb, 0]],
                            osem.at[b]).wait()

    idx_start(0)
    idx_start(1)
    idx_wait(0)
    gather_start(0, 0)
    def chunk_body(j, _):
      b = lax.rem(j, 2)
      b1 = lax.rem(j + 1, 2)
      @pl.when(j >= 1)
      def _():
        scatter_wait(j - 1, b1)
      @pl.when(j + 1 < NCHUNK)
      def _():
        idx_wait(j + 1)
        gather_start(j + 1, b1)
      gather_wait(j, b)
      scatter_start(j, b)
      @pl.when(j + 2 < NCHUNK)
      def _():
        idx_start(j + 2)
      return 0
    lax.fori_loop(0, NCHUNK, chunk_body, 0)
    scatter_wait(NCHUNK - 1, (NCHUNK - 1) % 2)
    plsc.subcore_barrier()

    def ecopy(k, _):
      r = zbase + k * CHUNK
      pltpu.sync_copy(acc.at[pl.ds(r, CHUNK)], rows.at[0])
      pltpu.sync_copy(rows.at[0], sums_hbm.at[c, pl.ds(r, CHUNK)])
      return 0
    lax.fori_loop(0, ROWS_PER_TILE // CHUNK, ecopy, 0)
    pltpu.sync_copy(cacc.at[pl.ds(zbase, ROWS_PER_TILE)], cbuf)
    pltpu.sync_copy(cbuf, counts_hbm.at[c, pl.ds(zbase, ROWS_PER_TILE)])

  return agg(x, idx_p)


def _tc_finish_body(p0_ref, p1_ref, c0_ref, c1_ref, x_ref, w_ref, o_ref):
  cnt = c0_ref[0] + c1_ref[0] + 1.0
  mean = (p0_ref[0] + p1_ref[0] + x_ref[...]) / cnt
  o_ref[...] = jnp.dot(mean, w_ref[...], preferred_element_type=jnp.float32)


def _tc_finish(sums, counts3, x, w):
  blk = 1000
  grid = N // blk
  return pl.pallas_call(
      _tc_finish_body,
      grid=(grid,),
      in_specs=[
          pl.BlockSpec((1, blk, D), lambda i: (0, i, 0)),
          pl.BlockSpec((1, blk, D), lambda i: (1, i, 0)),
          pl.BlockSpec((1, blk, 1), lambda i: (0, i, 0)),
          pl.BlockSpec((1, blk, 1), lambda i: (1, i, 0)),
          pl.BlockSpec((blk, D), lambda i: (i, 0)),
          pl.BlockSpec((D, D), lambda i: (0, 0)),
      ],
      out_specs=pl.BlockSpec((blk, D), lambda i: (i, 0)),
      out_shape=jax.ShapeDtypeStruct((N, D), jnp.float32),
  )(sums, sums, counts3, counts3, x, w)


def kernel(input_matrix, adjacency_coo_matrix, weights_matrix):
  x = input_matrix
  src = adjacency_coo_matrix[0].astype(jnp.int32)
  dst = adjacency_coo_matrix[1].astype(jnp.int32)
  e = src.shape[0]
  total = NTILES * PER_TILE
  npad = total - e
  pad_ar = jnp.arange(npad, dtype=jnp.int32)
  src_p = jnp.concatenate([src, N + (pad_ar % 16)]).reshape(NTILES, NCHUNK, CHUNK)
  dst_p = jnp.concatenate([dst, (pad_ar * 97) % N]).reshape(NTILES, NCHUNK, CHUNK)
  idx_p = jnp.stack([src_p, dst_p], axis=2)

  sums, counts = _sc_aggregate(x, idx_p)
  counts3 = counts.reshape(2, NROWS, 1)
  return _tc_finish(sums, counts3, x, weights_matrix)


# speedup vs baseline: 20.2595x; 1.0428x over previous
import functools

import jax
import jax.numpy as jnp
from jax import lax
from jax.experimental import pallas as pl
from jax.experimental.pallas import tpu as pltpu
from jax.experimental.pallas import tpu_sc as plsc

N = 10000
D = 128
NROWS = 10240
ROWS_PER_TILE = NROWS // 16
CHUNK = 128
NCHUNKS = 2500
NTILES = 32


def _sc_aggregate(x, adj3, zrows, zcnt):
  mesh = plsc.VectorSubcoreMesh(core_axis_name="c", subcore_axis_name="s")

  @functools.partial(
      pl.kernel,
      out_type=(
          jax.ShapeDtypeStruct((2, NROWS, D), jnp.float32),
          jax.ShapeDtypeStruct((2, NROWS), jnp.float32),
      ),
      mesh=mesh,
      scratch_types=[
          pltpu.VMEM((4, 2, CHUNK), jnp.int32),
          pltpu.VMEM((2, CHUNK, D), jnp.float32),
          pltpu.VMEM((CHUNK,), jnp.float32),
          pltpu.VMEM_SHARED((NROWS, D), jnp.float32),
          pltpu.VMEM_SHARED((NROWS,), jnp.float32),
          pltpu.SemaphoreType.DMA((4,)),
          pltpu.SemaphoreType.DMA((2,)),
          pltpu.SemaphoreType.DMA((2,)),
          pltpu.SemaphoreType.DMA((2,)),
      ],
  )
  def agg(x_hbm, adj_hbm, zrows_hbm, zcnt_hbm, sums_hbm, counts_hbm,
          idx_v, rows, ones_v, acc, cacc, isem, gsem, ssem, osem):
    c = lax.axis_index("c")
    s = lax.axis_index("s")
    wid = s * 2 + c
    zbase = s * ROWS_PER_TILE
    one16 = jnp.ones((16,), jnp.float32)
    nk = lax.select(wid < NCHUNKS % NTILES,
                    jnp.int32(NCHUNKS // NTILES + 1),
                    jnp.int32(NCHUNKS // NTILES))

    def zone(j, _):
      ones_v[pl.ds(j * 16, 16)] = one16
      return 0
    lax.fori_loop(0, CHUNK // 16, zone, 0)
    pltpu.sync_copy(zrows_hbm.at[pl.ds(zbase, ROWS_PER_TILE)],
                    acc.at[pl.ds(zbase, ROWS_PER_TILE)])
    pltpu.sync_copy(zcnt_hbm.at[pl.ds(zbase, ROWS_PER_TILE)],
                    cacc.at[pl.ds(zbase, ROWS_PER_TILE)])
    plsc.subcore_barrier()

    def idx_start(k):
      kb = lax.rem(k, 4)
      q = wid + k * NTILES
      pltpu.async_copy(adj_hbm.at[0, q], idx_v.at[kb, 0], isem.at[kb])
      pltpu.async_copy(adj_hbm.at[1, q], idx_v.at[kb, 1], isem.at[kb])
    def idx_wait(k):
      kb = lax.rem(k, 4)
      q = wid + k * NTILES
      pltpu.make_async_copy(adj_hbm.at[0, q], idx_v.at[kb, 0],
                            isem.at[kb]).wait()
      pltpu.make_async_copy(adj_hbm.at[1, q], idx_v.at[kb, 1],
                            isem.at[kb]).wait()
    def gather_start(k, b):
      pltpu.async_copy(x_hbm.at[idx_v.at[lax.rem(k, 4), 1]], rows.at[b],
                       gsem.at[b])
    def gather_wait(k, b):
      pltpu.make_async_copy(x_hbm.at[idx_v.at[lax.rem(k, 4), 1]], rows.at[b],
                            gsem.at[b]).wait()
    def scatter_start(k, b):
      kb = lax.rem(k, 4)
      pltpu.async_copy(rows.at[b], acc.at[idx_v.at[kb, 0]], ssem.at[b],
                       add=True)
      pltpu.async_copy(ones_v, cacc.at[idx_v.at[kb, 0]], osem.at[b],
                       add=True)
    def scatter_wait(k, b):
      kb = lax.rem(k, 4)
      pltpu.make_async_copy(rows.at[b], acc.at[idx_v.at[kb, 0]],
                            ssem.at[b]).wait()
      pltpu.make_async_copy(ones_v, cacc.at[idx_v.at[kb, 0]],
                            osem.at[b]).wait()

    idx_start(0)
    idx_start(1)
    idx_wait(0)
    gather_start(0, 0)
    def chunk_body(k, _):
      b = lax.rem(k, 2)
      b1 = lax.rem(k + 1, 2)
      @pl.when(k >= 1)
      def _():
        scatter_wait(k - 1, b1)
      @pl.when(k + 1 < nk)
      def _():
        idx_wait(k + 1)
        gather_start(k + 1, b1)
      gather_wait(k, b)
      scatter_start(k, b)
      @pl.when(k + 2 < nk)
      def _():
        idx_start(k + 2)
      return 0
    lax.fori_loop(0, nk, chunk_body, 0)
    scatter_wait(nk - 1, lax.rem(nk - 1, 2))
    plsc.subcore_barrier()

    pltpu.sync_copy(acc.at[pl.ds(zbase, ROWS_PER_TILE)],
                    sums_hbm.at[c, pl.ds(zbase, ROWS_PER_TILE)])
    pltpu.sync_copy(cacc.at[pl.ds(zbase, ROWS_PER_TILE)],
                    counts_hbm.at[c, pl.ds(zbase, ROWS_PER_TILE)])

  return agg(x, adj3, zrows, zcnt)


def _tc_finish_body(p0_ref, p1_ref, c0_ref, c1_ref, x_ref, w_ref, o_ref):
  cnt = c0_ref[0] + c1_ref[0] + 1.0
  mean = (p0_ref[0] + p1_ref[0] + x_ref[...]) / cnt
  o_ref[...] = jnp.dot(mean, w_ref[...], preferred_element_type=jnp.float32)


def _tc_finish(sums, counts3, x, w):
  blk = 1000
  grid = N // blk
  return pl.pallas_call(
      _tc_finish_body,
      grid=(grid,),
      in_specs=[
          pl.BlockSpec((1, blk, D), lambda i: (0, i, 0)),
          pl.BlockSpec((1, blk, D), lambda i: (1, i, 0)),
          pl.BlockSpec((1, blk, 1), lambda i: (0, i, 0)),
          pl.BlockSpec((1, blk, 1), lambda i: (1, i, 0)),
          pl.BlockSpec((blk, D), lambda i: (i, 0)),
          pl.BlockSpec((D, D), lambda i: (0, 0)),
      ],
      out_specs=pl.BlockSpec((blk, D), lambda i: (i, 0)),
      out_shape=jax.ShapeDtypeStruct((N, D), jnp.float32),
  )(sums, sums, counts3, counts3, x, w)


def kernel(input_matrix, adjacency_coo_matrix, weights_matrix):
  x = input_matrix
  e = adjacency_coo_matrix.shape[1]
  assert e == NCHUNKS * CHUNK
  adj3 = adjacency_coo_matrix.astype(jnp.int32).reshape(2, NCHUNKS, CHUNK)
  zrows = jnp.zeros((NROWS, D), jnp.float32)
  zcnt = jnp.zeros((NROWS,), jnp.float32)

  sums, counts = _sc_aggregate(x, adj3, zrows, zcnt)
  counts3 = counts.reshape(2, NROWS, 1)
  return _tc_finish(sums, counts3, x, weights_matrix)


# speedup vs baseline: 21.1297x; 1.0430x over previous
import functools

import jax
import jax.numpy as jnp
import numpy as np
from jax import lax
from jax.experimental import pallas as pl
from jax.experimental.pallas import tpu as pltpu
from jax.experimental.pallas import tpu_sc as plsc

N = 10000
D = 128
NROWS = 10240
ROWS_PER_TILE = NROWS // 16
CHUNK = 128
NCHUNKS = 2500
NTILES = 32


def _sc_aggregate(x, adj3, zrows, zcnt):
  mesh = plsc.VectorSubcoreMesh(core_axis_name="c", subcore_axis_name="s")

  @functools.partial(
      pl.kernel,
      out_type=(
          jax.ShapeDtypeStruct((2, NROWS, D), jnp.float32),
          jax.ShapeDtypeStruct((2, NROWS), jnp.float32),
      ),
      mesh=mesh,
      scratch_types=[
          pltpu.VMEM((4, 2, CHUNK), jnp.int32),
          pltpu.VMEM((2, CHUNK, D), jnp.float32),
          pltpu.VMEM((CHUNK,), jnp.float32),
          pltpu.VMEM_SHARED((NROWS, D), jnp.float32),
          pltpu.VMEM_SHARED((NROWS,), jnp.float32),
          pltpu.SemaphoreType.DMA((4,)),
          pltpu.SemaphoreType.DMA((2,)),
          pltpu.SemaphoreType.DMA((2,)),
          pltpu.SemaphoreType.DMA((2,)),
      ],
  )
  def agg(x_hbm, adj_hbm, zrows_hbm, zcnt_hbm, sums_hbm, counts_hbm,
          idx_v, rows, ones_v, acc, cacc, isem, gsem, ssem, osem):
    c = lax.axis_index("c")
    s = lax.axis_index("s")
    wid = s * 2 + c
    zbase = s * ROWS_PER_TILE
    one16 = jnp.ones((16,), jnp.float32)
    nk = lax.select(wid < NCHUNKS % NTILES,
                    jnp.int32(NCHUNKS // NTILES + 1),
                    jnp.int32(NCHUNKS // NTILES))

    def zone(j, _):
      ones_v[pl.ds(j * 16, 16)] = one16
      return 0
    lax.fori_loop(0, CHUNK // 16, zone, 0)
    pltpu.sync_copy(zrows_hbm.at[pl.ds(zbase, ROWS_PER_TILE)],
                    acc.at[pl.ds(zbase, ROWS_PER_TILE)])
    pltpu.sync_copy(zcnt_hbm.at[pl.ds(zbase, ROWS_PER_TILE)],
                    cacc.at[pl.ds(zbase, ROWS_PER_TILE)])
    plsc.subcore_barrier()

    def idx_start(k):
      kb = lax.rem(k, 4)
      off = (wid + k * NTILES) * CHUNK
      pltpu.async_copy(adj_hbm.at[0, pl.ds(off, CHUNK)], idx_v.at[kb, 0],
                       isem.at[kb])
      pltpu.async_copy(adj_hbm.at[1, pl.ds(off, CHUNK)], idx_v.at[kb, 1],
                       isem.at[kb])
    def idx_wait(k):
      kb = lax.rem(k, 4)
      off = (wid + k * NTILES) * CHUNK
      pltpu.make_async_copy(adj_hbm.at[0, pl.ds(off, CHUNK)], idx_v.at[kb, 0],
                            isem.at[kb]).wait()
      pltpu.make_async_copy(adj_hbm.at[1, pl.ds(off, CHUNK)], idx_v.at[kb, 1],
                            isem.at[kb]).wait()
    def gather_start(k, b):
      pltpu.async_copy(x_hbm.at[idx_v.at[lax.rem(k, 4), 1]], rows.at[b],
                       gsem.at[b])
    def gather_wait(k, b):
      pltpu.make_async_copy(x_hbm.at[idx_v.at[lax.rem(k, 4), 1]], rows.at[b],
                            gsem.at[b]).wait()
    def scatter_start(k, b):
      kb = lax.rem(k, 4)
      pltpu.async_copy(rows.at[b], acc.at[idx_v.at[kb, 0]], ssem.at[b],
                       add=True)
      pltpu.async_copy(ones_v, cacc.at[idx_v.at[kb, 0]], osem.at[b],
                       add=True)
    def scatter_wait(k, b):
      kb = lax.rem(k, 4)
      pltpu.make_async_copy(rows.at[b], acc.at[idx_v.at[kb, 0]],
                            ssem.at[b]).wait()
      pltpu.make_async_copy(ones_v, cacc.at[idx_v.at[kb, 0]],
                            osem.at[b]).wait()

    idx_start(0)
    idx_start(1)
    idx_wait(0)
    gather_start(0, 0)
    def chunk_body(k, _):
      b = lax.rem(k, 2)
      b1 = lax.rem(k + 1, 2)
      @pl.when(k >= 1)
      def _():
        scatter_wait(k - 1, b1)
      @pl.when(k + 1 < nk)
      def _():
        idx_wait(k + 1)
        gather_start(k + 1, b1)
      gather_wait(k, b)
      scatter_start(k, b)
      @pl.when(k + 2 < nk)
      def _():
        idx_start(k + 2)
      return 0
    lax.fori_loop(0, nk, chunk_body, 0)
    scatter_wait(nk - 1, lax.rem(nk - 1, 2))
    plsc.subcore_barrier()

    pltpu.sync_copy(acc.at[pl.ds(zbase, ROWS_PER_TILE)],
                    sums_hbm.at[c, pl.ds(zbase, ROWS_PER_TILE)])
    pltpu.sync_copy(cacc.at[pl.ds(zbase, ROWS_PER_TILE)],
                    counts_hbm.at[c, pl.ds(zbase, ROWS_PER_TILE)])

  return agg(x, adj3, zrows, zcnt)


def _tc_finish_body(p0_ref, p1_ref, c0_ref, c1_ref, x_ref, w_ref, o_ref):
  cnt = c0_ref[0] + c1_ref[0] + 1.0
  mean = (p0_ref[0] + p1_ref[0] + x_ref[...]) / cnt
  o_ref[...] = jnp.dot(mean, w_ref[...], preferred_element_type=jnp.float32)


def _tc_finish(sums, counts3, x, w):
  blk = 2000
  grid = N // blk
  return pl.pallas_call(
      _tc_finish_body,
      grid=(grid,),
      in_specs=[
          pl.BlockSpec((1, blk, D), lambda i: (0, i, 0)),
          pl.BlockSpec((1, blk, D), lambda i: (1, i, 0)),
          pl.BlockSpec((1, blk, 1), lambda i: (0, i, 0)),
          pl.BlockSpec((1, blk, 1), lambda i: (1, i, 0)),
          pl.BlockSpec((blk, D), lambda i: (i, 0)),
          pl.BlockSpec((D, D), lambda i: (0, 0)),
      ],
      out_specs=pl.BlockSpec((blk, D), lambda i: (i, 0)),
      out_shape=jax.ShapeDtypeStruct((N, D), jnp.float32),
  )(sums, sums, counts3, counts3, x, w)


_ZROWS = np.zeros((NROWS, D), np.float32)
_ZCNT = np.zeros((NROWS,), np.float32)


def kernel(input_matrix, adjacency_coo_matrix, weights_matrix):
  x = input_matrix
  e = adjacency_coo_matrix.shape[1]
  assert e == NCHUNKS * CHUNK
  adj3 = adjacency_coo_matrix.astype(jnp.int32)
  zrows = _ZROWS
  zcnt = _ZCNT

  sums, counts = _sc_aggregate(x, adj3, zrows, zcnt)
  counts3 = counts.reshape(2, NROWS, 1)
  return _tc_finish(sums, counts3, x, weights_matrix)


# speedup vs baseline: 22.4063x; 1.0604x over previous
import functools

import jax
import jax.numpy as jnp
import numpy as np
from jax import lax
from jax.experimental import pallas as pl
from jax.experimental.pallas import tpu as pltpu
from jax.experimental.pallas import tpu_sc as plsc

N = 10000
D = 128
RPT = 632
RPT_LAST = N - 15 * RPT
CNT_ROWS = 10112
CPT = 640
CPT_LAST = CNT_ROWS - 15 * CPT
CHUNK = 128
NCHUNKS = 2500
NTILES = 32


def _sc_aggregate(x, adj3, zrows, zcnt):
  mesh = plsc.VectorSubcoreMesh(core_axis_name="c", subcore_axis_name="s")

  @functools.partial(
      pl.kernel,
      out_type=(
          jax.ShapeDtypeStruct((2, N, D), jnp.float32),
          jax.ShapeDtypeStruct((2 * CNT_ROWS,), jnp.float32),
      ),
      mesh=mesh,
      scratch_types=[
          pltpu.VMEM((3, CHUNK), jnp.int32),
          pltpu.VMEM((4, CHUNK), jnp.int32),
          pltpu.VMEM((3, CHUNK, D), jnp.float32),
          pltpu.VMEM((CHUNK,), jnp.float32),
          pltpu.VMEM_SHARED((N, D), jnp.float32),
          pltpu.VMEM_SHARED((CNT_ROWS,), jnp.float32),
          pltpu.SemaphoreType.DMA((3,)),
          pltpu.SemaphoreType.DMA((4,)),
          pltpu.SemaphoreType.DMA((3,)),
          pltpu.SemaphoreType.DMA((3,)),
          pltpu.SemaphoreType.DMA((3,)),
      ],
  )
  def agg(x_hbm, adj_hbm, zrows_hbm, zcnt_hbm, sums_hbm, counts_hbm,
          src_v, dst_v, rows, ones_v, acc, cacc,
          sisem, disem, gsem, ssem, osem):
    c = lax.axis_index("c")
    s = lax.axis_index("s")
    wid = s * 2 + c
    one16 = jnp.ones((16,), jnp.float32)
    nk = lax.select(wid < NCHUNKS % NTILES,
                    jnp.int32(NCHUNKS // NTILES + 1),
                    jnp.int32(NCHUNKS // NTILES))

    def zone(j, _):
      ones_v[pl.ds(j * 16, 16)] = one16
      return 0
    lax.fori_loop(0, CHUNK // 16, zone, 0)
    @pl.when(s < 15)
    def _():
      pltpu.sync_copy(zrows_hbm, acc.at[pl.ds(s * RPT, RPT)])
      pltpu.sync_copy(zcnt_hbm, cacc.at[pl.ds(s * CPT, CPT)])
    @pl.when(s == 15)
    def _():
      pltpu.sync_copy(zrows_hbm.at[pl.ds(0, RPT_LAST)],
                      acc.at[pl.ds(15 * RPT, RPT_LAST)])
      pltpu.sync_copy(zcnt_hbm.at[pl.ds(0, CPT_LAST)],
                      cacc.at[pl.ds(15 * CPT, CPT_LAST)])
    plsc.subcore_barrier()

    def src_start(k):
      kb = lax.rem(k, 3)
      off = (wid + k * NTILES) * CHUNK
      pltpu.async_copy(adj_hbm.at[0, pl.ds(off, CHUNK)], src_v.at[kb],
                       sisem.at[kb])
    def src_wait(k):
      kb = lax.rem(k, 3)
      off = (wid + k * NTILES) * CHUNK
      pltpu.make_async_copy(adj_hbm.at[0, pl.ds(off, CHUNK)], src_v.at[kb],
                            sisem.at[kb]).wait()
    def dst_start(k):
      kb = lax.rem(k, 4)
      off = (wid + k * NTILES) * CHUNK
      pltpu.async_copy(adj_hbm.at[1, pl.ds(off, CHUNK)], dst_v.at[kb],
                       disem.at[kb])
    def dst_wait(k):
      kb = lax.rem(k, 4)
      off = (wid + k * NTILES) * CHUNK
      pltpu.make_async_copy(adj_hbm.at[1, pl.ds(off, CHUNK)], dst_v.at[kb],
                            disem.at[kb]).wait()
    def gather_start(k, b):
      pltpu.async_copy(x_hbm.at[dst_v.at[lax.rem(k, 4)]], rows.at[b],
                       gsem.at[b])
    def gather_wait(k, b):
      pltpu.make_async_copy(x_hbm.at[dst_v.at[lax.rem(k, 4)]], rows.at[b],
                            gsem.at[b]).wait()
    def scatter_start(k, b):
      kb = lax.rem(k, 3)
      pltpu.async_copy(rows.at[b], acc.at[src_v.at[kb]], ssem.at[b],
                       add=True)
      pltpu.async_copy(ones_v, cacc.at[src_v.at[kb]], osem.at[b],
                       add=True)
    def scatter_wait(k, b):
      kb = lax.rem(k, 3)
      pltpu.make_async_copy(rows.at[b], acc.at[src_v.at[kb]],
                            ssem.at[b]).wait()
      pltpu.make_async_copy(ones_v, cacc.at[src_v.at[kb]],
                            osem.at[b]).wait()

    src_start(0)
    src_start(1)
    dst_start(0)
    dst_start(1)
    dst_start(2)
    dst_wait(0)
    gather_start(0, 0)
    dst_wait(1)
    gather_start(1, 1)
    def chunk_body(k, _):
      b = lax.rem(k, 3)
      b2 = lax.rem(k + 2, 3)
      @pl.when(k >= 1)
      def _():
        scatter_wait(k - 1, b2)
      @pl.when(k + 2 < nk)
      def _():
        dst_wait(k + 2)
        gather_start(k + 2, b2)
      gather_wait(k, b)
      src_wait(k)
      scatter_start(k, b)
      @pl.when(k + 2 < nk)
      def _():
        src_start(k + 2)
      @pl.when(k + 3 < nk)
      def _():
        dst_start(k + 3)
      return 0
    lax.fori_loop(0, nk, chunk_body, 0)
    scatter_wait(nk - 1, lax.rem(nk - 1, 3))
    plsc.subcore_barrier()

    @pl.when(s < 15)
    def _():
      pltpu.sync_copy(acc.at[pl.ds(s * RPT, RPT)],
                      sums_hbm.at[c, pl.ds(s * RPT, RPT)])
      pltpu.sync_copy(cacc.at[pl.ds(s * CPT, CPT)],
                      counts_hbm.at[pl.ds(c * CNT_ROWS + s * CPT, CPT)])
    @pl.when(s == 15)
    def _():
      pltpu.sync_copy(acc.at[pl.ds(15 * RPT, RPT_LAST)],
                      sums_hbm.at[c, pl.ds(15 * RPT, RPT_LAST)])
      pltpu.sync_copy(cacc.at[pl.ds(15 * CPT, CPT_LAST)],
                      counts_hbm.at[pl.ds(c * CNT_ROWS + 15 * CPT, CPT_LAST)])

  return agg(x, adj3, zrows, zcnt)


def _tc_finish_body(p_ref, c_ref, x_ref, w_ref, o_ref):
  cnt = c_ref[0] + c_ref[1] + 1.0
  mean = (p_ref[0] + p_ref[1] + x_ref[...]) / cnt
  o_ref[...] = jnp.dot(mean, w_ref[...], preferred_element_type=jnp.float32)


def _tc_finish(sums, counts3, x, w):
  blk = 2000
  grid = N // blk
  return pl.pallas_call(
      _tc_finish_body,
      grid=(grid,),
      in_specs=[
          pl.BlockSpec((2, blk, D), lambda i: (0, i, 0)),
          pl.BlockSpec((2, blk, 1), lambda i: (0, i, 0)),
          pl.BlockSpec((blk, D), lambda i: (i, 0)),
          pl.BlockSpec((D, D), lambda i: (0, 0)),
      ],
      out_specs=pl.BlockSpec((blk, D), lambda i: (i, 0)),
      out_shape=jax.ShapeDtypeStruct((N, D), jnp.float32),
  )(sums, counts3, x, w)


_ZROWS = np.zeros((RPT, D), np.float32)
_ZCNT = np.zeros((CPT,), np.float32)


def kernel(input_matrix, adjacency_coo_matrix, weights_matrix):
  x = input_matrix
  e = adjacency_coo_matrix.shape[1]
  assert e == NCHUNKS * CHUNK
  adj3 = adjacency_coo_matrix.astype(jnp.int32)
  zrows = _ZROWS
  zcnt = _ZCNT

  sums, counts = _sc_aggregate(x, adj3, zrows, zcnt)
  counts3 = counts.reshape(2, CNT_ROWS, 1)
  return _tc_finish(sums, counts3, x, weights_matrix)


# speedup vs baseline: 24.0592x; 1.0738x over previous
import functools

import jax
import jax.numpy as jnp
import numpy as np
from jax import lax
from jax.experimental import pallas as pl
from jax.experimental.pallas import tpu as pltpu
from jax.experimental.pallas import tpu_sc as plsc

N = 10000
D = 128
RPT = 632
RPT_LAST = N - 15 * RPT
CNT_ROWS = 10112
CPT = 640
CPT_LAST = CNT_ROWS - 15 * CPT
CHUNK = 128
NCHUNKS = 2500
NTILES = 32


def _sc_aggregate(x, adj3, zrows, zcnt):
  mesh = plsc.VectorSubcoreMesh(core_axis_name="c", subcore_axis_name="s")

  @functools.partial(
      pl.kernel,
      out_type=(
          jax.ShapeDtypeStruct((2, N, D), jnp.float32),
          jax.ShapeDtypeStruct((2 * CNT_ROWS,), jnp.float32),
      ),
      mesh=mesh,
      scratch_types=[
          pltpu.VMEM((3, CHUNK), jnp.int32),
          pltpu.VMEM((4, CHUNK), jnp.int32),
          pltpu.VMEM((3, CHUNK, D), jnp.float32),
          pltpu.VMEM((CHUNK,), jnp.float32),
          pltpu.VMEM_SHARED((N, D), jnp.float32),
          pltpu.VMEM_SHARED((CNT_ROWS,), jnp.float32),
          pltpu.SemaphoreType.DMA((3,)),
          pltpu.SemaphoreType.DMA((4,)),
          pltpu.SemaphoreType.DMA((3,)),
          pltpu.SemaphoreType.DMA((3,)),
          pltpu.SemaphoreType.DMA((3,)),
      ],
  )
  def agg(x_hbm, adj_hbm, zrows_hbm, zcnt_hbm, sums_hbm, counts_hbm,
          src_v, dst_v, rows, ones_v, acc, cacc,
          sisem, disem, gsem, ssem, osem):
    c = lax.axis_index("c")
    s = lax.axis_index("s")
    wid = s * 2 + c
    one16 = jnp.ones((16,), jnp.float32)
    nk = lax.select(wid < NCHUNKS % NTILES,
                    jnp.int32(NCHUNKS // NTILES + 1),
                    jnp.int32(NCHUNKS // NTILES))

    def zone(j, _):
      ones_v[pl.ds(j * 16, 16)] = one16
      return 0
    lax.fori_loop(0, CHUNK // 16, zone, 0)
    @pl.when(s < 15)
    def _():
      pltpu.sync_copy(zrows_hbm, acc.at[pl.ds(s * RPT, RPT)])
      pltpu.sync_copy(zcnt_hbm, cacc.at[pl.ds(s * CPT, CPT)])
    @pl.when(s == 15)
    def _():
      pltpu.sync_copy(zrows_hbm.at[pl.ds(0, RPT_LAST)],
                      acc.at[pl.ds(15 * RPT, RPT_LAST)])
      pltpu.sync_copy(zcnt_hbm.at[pl.ds(0, CPT_LAST)],
                      cacc.at[pl.ds(15 * CPT, CPT_LAST)])
    plsc.subcore_barrier()

    def src_start(k):
      kb = lax.rem(k, 3)
      off = (wid + k * NTILES) * CHUNK
      pltpu.async_copy(adj_hbm.at[0, pl.ds(off, CHUNK)], src_v.at[kb],
                       sisem.at[kb])
    def src_wait(k):
      kb = lax.rem(k, 3)
      off = (wid + k * NTILES) * CHUNK
      pltpu.make_async_copy(adj_hbm.at[0, pl.ds(off, CHUNK)], src_v.at[kb],
                            sisem.at[kb]).wait()
    def dst_start(k):
      kb = lax.rem(k, 4)
      off = (wid + k * NTILES) * CHUNK
      pltpu.async_copy(adj_hbm.at[1, pl.ds(off, CHUNK)], dst_v.at[kb],
                       disem.at[kb])
    def dst_wait(k):
      kb = lax.rem(k, 4)
      off = (wid + k * NTILES) * CHUNK
      pltpu.make_async_copy(adj_hbm.at[1, pl.ds(off, CHUNK)], dst_v.at[kb],
                            disem.at[kb]).wait()
    def gather_start(k, b):
      pltpu.async_copy(x_hbm.at[dst_v.at[lax.rem(k, 4)]], rows.at[b],
                       gsem.at[b])
    def gather_wait(k, b):
      pltpu.make_async_copy(x_hbm.at[dst_v.at[lax.rem(k, 4)]], rows.at[b],
                            gsem.at[b]).wait()
    def scatter_start(k, b):
      kb = lax.rem(k, 3)
      pltpu.async_copy(rows.at[b], acc.at[src_v.at[kb]], ssem.at[b],
                       add=True)
      pltpu.async_copy(ones_v, cacc.at[src_v.at[kb]], osem.at[b],
                       add=True)
    def scatter_wait(k, b):
      kb = lax.rem(k, 3)
      pltpu.make_async_copy(rows.at[b], acc.at[src_v.at[kb]],
                            ssem.at[b]).wait()
      pltpu.make_async_copy(ones_v, cacc.at[src_v.at[kb]],
                            osem.at[b]).wait()

    src_start(0)
    src_start(1)
    dst_start(0)
    dst_start(1)
    dst_start(2)
    dst_wait(0)
    gather_start(0, 0)
    dst_wait(1)
    gather_start(1, 1)
    def chunk_body(k, _):
      b = lax.rem(k, 3)
      b2 = lax.rem(k + 2, 3)
      @pl.when(k >= 1)
      def _():
        scatter_wait(k - 1, b2)
      @pl.when(k + 2 < nk)
      def _():
        dst_wait(k + 2)
        gather_start(k + 2, b2)
      gather_wait(k, b)
      src_wait(k)
      scatter_start(k, b)
      @pl.when(k + 2 < nk)
      def _():
        src_start(k + 2)
      @pl.when(k + 3 < nk)
      def _():
        dst_start(k + 3)
      return 0
    lax.fori_loop(0, nk, chunk_body, 0)
    scatter_wait(nk - 1, lax.rem(nk - 1, 3))
    plsc.subcore_barrier()

    @pl.when(s < 15)
    def _():
      pltpu.sync_copy(acc.at[pl.ds(s * RPT, RPT)],
                      sums_hbm.at[c, pl.ds(s * RPT, RPT)])
      pltpu.sync_copy(cacc.at[pl.ds(s * CPT, CPT)],
                      counts_hbm.at[pl.ds(c * CNT_ROWS + s * CPT, CPT)])
    @pl.when(s == 15)
    def _():
      pltpu.sync_copy(acc.at[pl.ds(15 * RPT, RPT_LAST)],
                      sums_hbm.at[c, pl.ds(15 * RPT, RPT_LAST)])
      pltpu.sync_copy(cacc.at[pl.ds(15 * CPT, CPT_LAST)],
                      counts_hbm.at[pl.ds(c * CNT_ROWS + 15 * CPT, CPT_LAST)])

  return agg(x, adj3, zrows, zcnt)


def _tc_finish_body(p_ref, c_ref, x_ref, w_ref, o_ref):
  cpair = jnp.transpose(c_ref[...])
  cnt = cpair[:N, 0:1] + cpair[:N, 1:2] + 1.0
  mean = (p_ref[0] + p_ref[1] + x_ref[...]) / cnt
  o_ref[...] = jnp.dot(mean, w_ref[...], preferred_element_type=jnp.float32)


def _tc_finish(sums, counts2, x, w):
  return pl.pallas_call(
      _tc_finish_body,
      grid=(1,),
      in_specs=[
          pl.BlockSpec((2, N, D), lambda i: (0, 0, 0)),
          pl.BlockSpec((2, CNT_ROWS), lambda i: (0, 0)),
          pl.BlockSpec((N, D), lambda i: (0, 0)),
          pl.BlockSpec((D, D), lambda i: (0, 0)),
      ],
      out_specs=pl.BlockSpec((N, D), lambda i: (0, 0)),
      out_shape=jax.ShapeDtypeStruct((N, D), jnp.float32),
  )(sums, counts2, x, w)


_ZROWS = np.zeros((RPT, D), np.float32)
_ZCNT = np.zeros((CPT,), np.float32)


def kernel(input_matrix, adjacency_coo_matrix, weights_matrix):
  x = input_matrix
  e = adjacency_coo_matrix.shape[1]
  assert e == NCHUNKS * CHUNK
  adj3 = adjacency_coo_matrix.astype(jnp.int32)
  zrows = _ZROWS
  zcnt = _ZCNT

  sums, counts = _sc_aggregate(x, adj3, zrows, zcnt)
  counts2 = counts.reshape(2, CNT_ROWS)
  return _tc_finish(sums, counts2, x, weights_matrix)
